# Initial kernel scaffold; baseline (speedup 1.0000x reference)
import jax
import jax.numpy as jnp
from jax.experimental import pallas as pl


def kernel(targets, contexts, neg_idx, rho, alpha):
    raise NotImplementedError("write your pallas kernel here")


# speedup vs baseline: 2.9788x; 2.9788x over previous
import functools
import math

import jax
import jax.numpy as jnp
from jax import lax
from jax.experimental import pallas as pl
from jax.experimental.pallas import tpu as pltpu
from jax.experimental.pallas import tpu_sc as plsc

V = 100000
D = 64
CS = 8
NS = 20
B = 16384
SIGMA = 0.1

NC = 2
NSC = 16
NW = NC * NSC
BPW = B // NW
CHUNK = 32
NCHUNK = BPW // CHUNK
NJ = NS + 1


NDOT = CHUNK * NJ
NGRP = NDOT // 16


def _sc_eta_body(tgt_hbm, ctx_hbm, neg_hbm, rho_hbm, alpha_hbm, out_hbm,
                 tgt_idx, ctx_idx, neg_idx_v, t_rows, a_rows, n_rows,
                 part_t, eta_v, sem_a, sem_t, sem_n):
    wid = lax.axis_index("s") * NC + lax.axis_index("c")
    lanes = lax.iota(jnp.int32, 16)

    def chunk_body(i, _):
        base = wid * BPW + i * CHUNK
        pltpu.sync_copy(tgt_hbm.at[pl.ds(base, CHUNK)], tgt_idx)
        pltpu.sync_copy(ctx_hbm.at[pl.ds(base * CS, CHUNK * CS)], ctx_idx)
        pltpu.sync_copy(neg_hbm.at[pl.ds(base * NS, CHUNK * NS)], neg_idx_v)
        cp_a = pltpu.async_copy(alpha_hbm.at[ctx_idx], a_rows, sem_a)
        cp_t = pltpu.async_copy(rho_hbm.at[tgt_idx], t_rows, sem_t)
        cp_n = pltpu.async_copy(rho_hbm.at[neg_idx_v], n_rows, sem_n)
        cp_a.wait()
        cp_t.wait()
        cp_n.wait()

        def b_body(b, _):
            acc = []
            for k in range(D // 16):
                s = a_rows[b * CS + 0, pl.ds(k * 16, 16)]
                for c in range(1, CS):
                    s = s + a_rows[b * CS + c, pl.ds(k * 16, 16)]
                acc.append(s)
            dotbase = b * NJ
            p = t_rows[b, pl.ds(0, 16)] * acc[0]
            for k in range(1, D // 16):
                p = p + t_rows[b, pl.ds(k * 16, 16)] * acc[k]
            plsc.store_scatter(
                part_t, [lanes * NDOT + dotbase], p)
            for j in range(NS):
                q = n_rows[b * NS + j, pl.ds(0, 16)] * acc[0]
                for k in range(1, D // 16):
                    q = q + n_rows[b * NS + j, pl.ds(k * 16, 16)] * acc[k]
                plsc.store_scatter(
                    part_t, [lanes * NDOT + (dotbase + 1 + j)], q)
            return 0

        lax.fori_loop(0, CHUNK, b_body, 0)

        def g_body(g, _):
            s = part_t[pl.ds(g * 16, 16)]
            for r in range(1, 16):
                s = s + part_t[pl.ds(r * NDOT + g * 16, 16)]
            eta_v[pl.ds(g * 16, 16)] = s
            return 0

        lax.fori_loop(0, NGRP, g_body, 0)
        pltpu.sync_copy(eta_v, out_hbm.at[pl.ds(base * NJ, NDOT)])
        return 0

    lax.fori_loop(0, NCHUNK, chunk_body, 0)


_sc_etas = pl.kernel(
    _sc_eta_body,
    out_type=jax.ShapeDtypeStruct((B * NJ,), jnp.float32),
    mesh=plsc.VectorSubcoreMesh(core_axis_name="c", subcore_axis_name="s",
                                num_cores=NC, num_subcores=NSC),
    compiler_params=pltpu.CompilerParams(needs_layout_passes=False,
                                         use_tc_tiling_on_sc=False),
    scratch_types=[
        pltpu.VMEM((CHUNK,), jnp.int32),
        pltpu.VMEM((CHUNK * CS,), jnp.int32),
        pltpu.VMEM((CHUNK * NS,), jnp.int32),
        pltpu.VMEM((CHUNK, D), jnp.float32),
        pltpu.VMEM((CHUNK * CS, D), jnp.float32),
        pltpu.VMEM((CHUNK * NS, D), jnp.float32),
        pltpu.VMEM((16 * NDOT,), jnp.float32),
        pltpu.VMEM((NDOT,), jnp.float32),
        pltpu.SemaphoreType.DMA,
        pltpu.SemaphoreType.DMA,
        pltpu.SemaphoreType.DMA,
    ],
)


ROWS_PER_STEP = 2000
NSTEP = V // ROWS_PER_STEP
_PRIOR_CONST = 2.0 * V * D * (-math.log(SIGMA) - 0.5 * math.log(2.0 * math.pi))


def _tc_loss_body(eta_ref, rho_ref, alpha_ref, out_ref, acc_ref):
    step = pl.program_id(0)

    @pl.when(step == 0)
    def _():
        eta = eta_ref[...]
        col = lax.broadcasted_iota(jnp.int32, eta.shape, 1)
        signed = jnp.where(col == 0, eta, -eta)
        acc_ref[0] = jnp.sum(jax.nn.log_sigmoid(signed))
        acc_ref[1] = 0.0

    r = rho_ref[...]
    a = alpha_ref[...]
    acc_ref[1] += jnp.sum(r * r) + jnp.sum(a * a)

    @pl.when(step == NSTEP - 1)
    def _():
        lp_prior = -0.5 / (SIGMA * SIGMA) * acc_ref[1] + _PRIOR_CONST
        out_ref[0, 0] = -(acc_ref[0] + lp_prior)


_tc_loss = pl.pallas_call(
    _tc_loss_body,
    grid=(NSTEP,),
    in_specs=[
        pl.BlockSpec((B, NJ), lambda i: (0, 0)),
        pl.BlockSpec((ROWS_PER_STEP, D), lambda i: (i, 0)),
        pl.BlockSpec((ROWS_PER_STEP, D), lambda i: (i, 0)),
    ],
    out_specs=pl.BlockSpec(memory_space=pltpu.SMEM),
    out_shape=jax.ShapeDtypeStruct((1, 1), jnp.float32),
    scratch_shapes=[pltpu.SMEM((2,), jnp.float32)],
)


def kernel(targets, contexts, neg_idx, rho, alpha):
    ctx_flat = contexts.reshape(-1)
    neg_flat = neg_idx.reshape(-1)
    eta = _sc_etas(targets, ctx_flat, neg_flat, rho, alpha)
    loss = _tc_loss(eta.reshape(B, NJ), rho, alpha)
    return loss[0, 0]


# speedup vs baseline: 3.9638x; 1.3307x over previous
import math

import jax
import jax.numpy as jnp
from jax import lax
from jax.experimental import pallas as pl
from jax.experimental.pallas import tpu as pltpu
from jax.experimental.pallas import tpu_sc as plsc

V = 100000
D = 64
CS = 8
NS = 20
B = 16384
SIGMA = 0.1

NC = 2
NSC = 16
NW = NC * NSC
BPW = B // NW
CHUNK = 16
NCHUNK = BPW // CHUNK
NJ = NS + 1
NDOT = CHUNK * NJ
NGRP = NDOT // 16


def _sc_eta_body(tgt_hbm, ctx_hbm, neg_hbm, rho_hbm, alpha_hbm, out_hbm,
                 tgt_idx, ctx_idx, neg_idx_v, t_rows, a_rows, n_rows,
                 part_t, eta_all, sems):
    wid = lax.axis_index("s") * NC + lax.axis_index("c")
    lanes = lax.iota(jnp.int32, 16)

    pltpu.sync_copy(tgt_hbm.at[pl.ds(wid * BPW, BPW)], tgt_idx)
    pltpu.sync_copy(ctx_hbm.at[pl.ds(wid * BPW * CS, BPW * CS)], ctx_idx)
    pltpu.sync_copy(neg_hbm.at[pl.ds(wid * BPW * NS, BPW * NS)], neg_idx_v)

    def issue(c, s):
        cp_a = pltpu.async_copy(
            alpha_hbm.at[ctx_idx.at[pl.ds(c * CHUNK * CS, CHUNK * CS)]],
            a_rows.at[s], sems.at[s, 0])
        cp_t = pltpu.async_copy(
            rho_hbm.at[tgt_idx.at[pl.ds(c * CHUNK, CHUNK)]],
            t_rows.at[s], sems.at[s, 1])
        cp_n = pltpu.async_copy(
            rho_hbm.at[neg_idx_v.at[pl.ds(c * CHUNK * NS, CHUNK * NS)]],
            n_rows.at[s], sems.at[s, 2])
        return cp_a, cp_t, cp_n

    def wait_set(s):
        pltpu.make_async_copy(
            alpha_hbm.at[ctx_idx.at[pl.ds(0, CHUNK * CS)]],
            a_rows.at[s], sems.at[s, 0]).wait()
        pltpu.make_async_copy(
            rho_hbm.at[tgt_idx.at[pl.ds(0, CHUNK)]],
            t_rows.at[s], sems.at[s, 1]).wait()
        pltpu.make_async_copy(
            rho_hbm.at[neg_idx_v.at[pl.ds(0, CHUNK * NS)]],
            n_rows.at[s], sems.at[s, 2]).wait()

    def compute(c, s):
        def b_body(b, _):
            acc = []
            for k in range(D // 16):
                v = a_rows[s, b * CS + 0, pl.ds(k * 16, 16)]
                for cc in range(1, CS):
                    v = v + a_rows[s, b * CS + cc, pl.ds(k * 16, 16)]
                acc.append(v)
            dotbase = b * NJ
            p = t_rows[s, b, pl.ds(0, 16)] * acc[0]
            for k in range(1, D // 16):
                p = p + t_rows[s, b, pl.ds(k * 16, 16)] * acc[k]
            plsc.store_scatter(part_t, [lanes * NDOT + dotbase], p)
            for j in range(NS):
                q = n_rows[s, b * NS + j, pl.ds(0, 16)] * acc[0]
                for k in range(1, D // 16):
                    q = q + n_rows[s, b * NS + j, pl.ds(k * 16, 16)] * acc[k]
                plsc.store_scatter(
                    part_t, [lanes * NDOT + (dotbase + 1 + j)], q)
            return 0

        lax.fori_loop(0, CHUNK, b_body, 0, unroll=False)

        def g_body(g, _):
            v = part_t[pl.ds(g * 16, 16)]
            for r in range(1, 16):
                v = v + part_t[pl.ds(r * NDOT + g * 16, 16)]
            eta_all[pl.ds(c * NDOT + g * 16, 16)] = v
            return 0

        lax.fori_loop(0, NGRP, g_body, 0, unroll=False)

    issue(0, 0)

    def pair_body(pr, _):
        i0 = 2 * pr
        issue(i0 + 1, 1)
        wait_set(0)
        compute(i0, 0)

        @pl.when(i0 + 2 < NCHUNK)
        def _():
            issue(i0 + 2, 0)

        wait_set(1)
        compute(i0 + 1, 1)
        return 0

    lax.fori_loop(0, NCHUNK // 2, pair_body, 0, unroll=False)
    pltpu.sync_copy(eta_all, out_hbm.at[pl.ds(wid * BPW * NJ, BPW * NJ)])


_sc_etas = pl.kernel(
    _sc_eta_body,
    out_type=jax.ShapeDtypeStruct((B * NJ,), jnp.float32),
    mesh=plsc.VectorSubcoreMesh(core_axis_name="c", subcore_axis_name="s",
                                num_cores=NC, num_subcores=NSC),
    compiler_params=pltpu.CompilerParams(needs_layout_passes=False,
                                         use_tc_tiling_on_sc=False),
    scratch_types=[
        pltpu.VMEM((BPW,), jnp.int32),
        pltpu.VMEM((BPW * CS,), jnp.int32),
        pltpu.VMEM((BPW * NS,), jnp.int32),
        pltpu.VMEM((2, CHUNK, D), jnp.float32),
        pltpu.VMEM((2, CHUNK * CS, D), jnp.float32),
        pltpu.VMEM((2, CHUNK * NS, D), jnp.float32),
        pltpu.VMEM((16 * NDOT,), jnp.float32),
        pltpu.VMEM((BPW * NJ,), jnp.float32),
        pltpu.SemaphoreType.DMA((2, 3)),
    ],
)


ROWS_PER_STEP = 2000
NSTEP = V // ROWS_PER_STEP
_PRIOR_CONST = 2.0 * V * D * (-math.log(SIGMA) - 0.5 * math.log(2.0 * math.pi))


def _tc_loss_body(eta_ref, rho_ref, alpha_ref, out_ref, acc_ref):
    step = pl.program_id(0)

    @pl.when(step == 0)
    def _():
        eta = eta_ref[...]
        col = lax.broadcasted_iota(jnp.int32, eta.shape, 1)
        signed = jnp.where(col == 0, eta, -eta)
        acc_ref[0] = jnp.sum(jax.nn.log_sigmoid(signed))
        acc_ref[1] = 0.0

    r = rho_ref[...]
    a = alpha_ref[...]
    acc_ref[1] += jnp.sum(r * r) + jnp.sum(a * a)

    @pl.when(step == NSTEP - 1)
    def _():
        lp_prior = -0.5 / (SIGMA * SIGMA) * acc_ref[1] + _PRIOR_CONST
        out_ref[0, 0] = -(acc_ref[0] + lp_prior)


_tc_loss = pl.pallas_call(
    _tc_loss_body,
    grid=(NSTEP,),
    in_specs=[
        pl.BlockSpec((B, NJ), lambda i: (0, 0)),
        pl.BlockSpec((ROWS_PER_STEP, D), lambda i: (i, 0)),
        pl.BlockSpec((ROWS_PER_STEP, D), lambda i: (i, 0)),
    ],
    out_specs=pl.BlockSpec(memory_space=pltpu.SMEM),
    out_shape=jax.ShapeDtypeStruct((1, 1), jnp.float32),
    scratch_shapes=[pltpu.SMEM((2,), jnp.float32)],
)


def kernel(targets, contexts, neg_idx, rho, alpha):
    ctx_flat = contexts.reshape(-1)
    neg_flat = neg_idx.reshape(-1)
    eta = _sc_etas(targets, ctx_flat, neg_flat, rho, alpha)
    loss = _tc_loss(eta.reshape(B, NJ), rho, alpha)
    return loss[0, 0]


# speedup vs baseline: 5.6283x; 1.4199x over previous
import math

import jax
import jax.numpy as jnp
from jax import lax
from jax.experimental import pallas as pl
from jax.experimental.pallas import tpu as pltpu
from jax.experimental.pallas import tpu_sc as plsc

V = 100000
D = 64
CS = 8
NS = 20
B = 16384
SIGMA = 0.1

NC = 2
NSC = 16
NW = NC * NSC
BPW = B // NW
CHUNK = 16
NCHUNK = BPW // CHUNK
NJ = NS + 1
NDOT = CHUNK * NJ
NGRP = NDOT // 16


def _sc_eta_body(tgt_hbm, ctx_hbm, neg_hbm, rho_hbm, alpha_hbm, out_hbm,
                 tgt_idx, ctx_idx, neg_idx_v, t_rows, a_rows, n_rows,
                 part_t, eta_all, sems):
    wid = lax.axis_index("s") * NC + lax.axis_index("c")
    lanes = lax.iota(jnp.int32, 16)

    pltpu.sync_copy(tgt_hbm.at[pl.ds(wid * BPW, BPW)], tgt_idx)
    pltpu.sync_copy(ctx_hbm.at[pl.ds(wid * BPW * CS, BPW * CS)], ctx_idx)
    pltpu.sync_copy(neg_hbm.at[pl.ds(wid * BPW * NS, BPW * NS)], neg_idx_v)

    def issue(c, s):
        cp_a = pltpu.async_copy(
            alpha_hbm.at[ctx_idx.at[pl.ds(c * CHUNK * CS, CHUNK * CS)]],
            a_rows.at[s], sems.at[s, 0])
        cp_t = pltpu.async_copy(
            rho_hbm.at[tgt_idx.at[pl.ds(c * CHUNK, CHUNK)]],
            t_rows.at[s], sems.at[s, 1])
        cp_n = pltpu.async_copy(
            rho_hbm.at[neg_idx_v.at[pl.ds(c * CHUNK * NS, CHUNK * NS)]],
            n_rows.at[s], sems.at[s, 2])
        return cp_a, cp_t, cp_n

    def wait_set(s):
        pltpu.make_async_copy(
            alpha_hbm.at[ctx_idx.at[pl.ds(0, CHUNK * CS)]],
            a_rows.at[s], sems.at[s, 0]).wait()
        pltpu.make_async_copy(
            rho_hbm.at[tgt_idx.at[pl.ds(0, CHUNK)]],
            t_rows.at[s], sems.at[s, 1]).wait()
        pltpu.make_async_copy(
            rho_hbm.at[neg_idx_v.at[pl.ds(0, CHUNK * NS)]],
            n_rows.at[s], sems.at[s, 2]).wait()

    def compute(c, s):
        def b_body(b, _):
            acc = []
            for k in range(D // 16):
                v = a_rows[s, b * CS + 0, pl.ds(k * 16, 16)]
                for cc in range(1, CS):
                    v = v + a_rows[s, b * CS + cc, pl.ds(k * 16, 16)]
                acc.append(v)
            dotbase = b * NJ
            p = t_rows[s, b, pl.ds(0, 16)] * acc[0]
            for k in range(1, D // 16):
                p = p + t_rows[s, b, pl.ds(k * 16, 16)] * acc[k]
            plsc.store_scatter(part_t, [lanes * NDOT + dotbase], p)
            for j in range(NS):
                q = n_rows[s, b * NS + j, pl.ds(0, 16)] * acc[0]
                for k in range(1, D // 16):
                    q = q + n_rows[s, b * NS + j, pl.ds(k * 16, 16)] * acc[k]
                plsc.store_scatter(
                    part_t, [lanes * NDOT + (dotbase + 1 + j)], q)
            return 0

        lax.fori_loop(0, CHUNK, b_body, 0, unroll=False)

        def g_body(g, _):
            v = part_t[pl.ds(g * 16, 16)]
            for r in range(1, 16):
                v = v + part_t[pl.ds(r * NDOT + g * 16, 16)]
            eta_all[pl.ds(c * NDOT + g * 16, 16)] = v
            return 0

        lax.fori_loop(0, NGRP, g_body, 0, unroll=False)

    issue(0, 0)

    def pair_body(pr, _):
        i0 = 2 * pr
        issue(i0 + 1, 1)
        wait_set(0)
        compute(i0, 0)

        @pl.when(i0 + 2 < NCHUNK)
        def _():
            issue(i0 + 2, 0)

        wait_set(1)
        compute(i0 + 1, 1)
        return 0

    lax.fori_loop(0, NCHUNK // 2, pair_body, 0, unroll=False)
    pltpu.sync_copy(eta_all, out_hbm.at[pl.ds(wid * BPW * NJ, BPW * NJ)])


_sc_etas = pl.kernel(
    _sc_eta_body,
    out_type=jax.ShapeDtypeStruct((B * NJ,), jnp.float32),
    mesh=plsc.VectorSubcoreMesh(core_axis_name="c", subcore_axis_name="s",
                                num_cores=NC, num_subcores=NSC),
    compiler_params=pltpu.CompilerParams(needs_layout_passes=False,
                                         use_tc_tiling_on_sc=False),
    scratch_types=[
        pltpu.VMEM((BPW,), jnp.int32),
        pltpu.VMEM((BPW * CS,), jnp.int32),
        pltpu.VMEM((BPW * NS,), jnp.int32),
        pltpu.VMEM((2, CHUNK, D), jnp.float32),
        pltpu.VMEM((2, CHUNK * CS, D), jnp.float32),
        pltpu.VMEM((2, CHUNK * NS, D), jnp.float32),
        pltpu.VMEM((16 * NDOT,), jnp.float32),
        pltpu.VMEM((BPW * NJ,), jnp.float32),
        pltpu.SemaphoreType.DMA((2, 3)),
    ],
)


_PRIOR_CONST = 2.0 * V * D * (-math.log(SIGMA) - 0.5 * math.log(2.0 * math.pi))

PRIOR_COLS = 8192
PRIOR_STEPS = -(-V // PRIOR_COLS)


def _tc_prior_body(rho_t_ref, alpha_t_ref, out_ref, acc_ref):
    step = pl.program_id(0)

    @pl.when(step == 0)
    def _():
        acc_ref[0] = 0.0

    col = step * PRIOR_COLS + lax.broadcasted_iota(
        jnp.int32, (D, PRIOR_COLS), 1)
    mask = col < V
    r = jnp.where(mask, rho_t_ref[...], 0.0)
    a = jnp.where(mask, alpha_t_ref[...], 0.0)
    acc_ref[0] += jnp.sum(r * r) + jnp.sum(a * a)

    @pl.when(step == PRIOR_STEPS - 1)
    def _():
        out_ref[0, 0] = acc_ref[0]


_tc_prior = pl.pallas_call(
    _tc_prior_body,
    grid=(PRIOR_STEPS,),
    in_specs=[
        pl.BlockSpec((D, PRIOR_COLS), lambda i: (0, i)),
        pl.BlockSpec((D, PRIOR_COLS), lambda i: (0, i)),
    ],
    out_specs=pl.BlockSpec(memory_space=pltpu.SMEM),
    out_shape=jax.ShapeDtypeStruct((1, 1), jnp.float32),
    scratch_shapes=[pltpu.SMEM((1,), jnp.float32)],
)


ETA_ROWS = (B * NJ) // 1024


def _tc_logsig_body(eta_ref, out_ref):
    eta = eta_ref[...]
    row = lax.broadcasted_iota(jnp.int32, eta.shape, 0)
    col = lax.broadcasted_iota(jnp.int32, eta.shape, 1)
    flat = row * 1024 + col
    signed = jnp.where(flat % NJ == 0, eta, -eta)
    out_ref[0, 0] = jnp.sum(jax.nn.log_sigmoid(signed))


_tc_logsig = pl.pallas_call(
    _tc_logsig_body,
    out_specs=pl.BlockSpec(memory_space=pltpu.SMEM),
    out_shape=jax.ShapeDtypeStruct((1, 1), jnp.float32),
)


def kernel(targets, contexts, neg_idx, rho, alpha):
    ctx_flat = contexts.reshape(-1)
    neg_flat = neg_idx.reshape(-1)
    eta = _sc_etas(targets, ctx_flat, neg_flat, rho, alpha)
    sq = _tc_prior(rho.T, alpha.T)[0, 0]
    ll = _tc_logsig(eta.reshape(ETA_ROWS, 1024))[0, 0]
    return -(ll + (-0.5 / (SIGMA * SIGMA)) * sq + _PRIOR_CONST)
